# Initial kernel scaffold; baseline (speedup 1.0000x reference)
import jax
import jax.numpy as jnp
from jax.experimental import pallas as pl


def kernel(t, x, f):
    raise NotImplementedError("write your pallas kernel here")


# speedup vs baseline: 1.3762x; 1.3762x over previous
import jax
import jax.numpy as jnp
from jax.experimental import pallas as pl
from jax.experimental.pallas import tpu as pltpu

B = 16384
N = 4096
BR = 256


def _body(f_ref, x_ref, o_ref):
    f = f_ref[0]
    x = x_ref[...]
    mask = x != 0.0
    cnt = jnp.sum(mask.astype(jnp.float32), axis=1, keepdims=True)
    denom = jnp.where(cnt > 0.0, f * cnt, 1.0)
    val = jnp.where(cnt > 0.0, f / denom, 0.0)
    o_ref[...] = jnp.where(mask, jnp.broadcast_to(val, x.shape), 0.0)


def kernel(t, x, f):
    del t
    return pl.pallas_call(
        _body,
        grid=(B // BR,),
        in_specs=[
            pl.BlockSpec(memory_space=pltpu.SMEM),
            pl.BlockSpec((BR, N), lambda i: (i, 0)),
        ],
        out_specs=pl.BlockSpec((BR, N), lambda i: (i, 0)),
        out_shape=jax.ShapeDtypeStruct((B, N), jnp.float32),
    )(f, x)


# speedup vs baseline: 1.3993x; 1.0168x over previous
import jax
import jax.numpy as jnp
from jax.experimental import pallas as pl
from jax.experimental.pallas import tpu as pltpu

B = 16384
N = 4096
BR = 512


def _body(f_ref, x_ref, o_ref):
    f = f_ref[0]
    x = x_ref[...]
    mask = x != 0.0
    cnt = jnp.sum(mask.astype(jnp.float32), axis=1, keepdims=True)
    denom = jnp.where(cnt > 0.0, f * cnt, 1.0)
    val = jnp.where(cnt > 0.0, f / denom, 0.0)
    o_ref[...] = jnp.where(mask, jnp.broadcast_to(val, x.shape), 0.0)


def kernel(t, x, f):
    del t
    return pl.pallas_call(
        _body,
        grid=(B // BR,),
        in_specs=[
            pl.BlockSpec(memory_space=pltpu.SMEM),
            pl.BlockSpec((BR, N), lambda i: (i, 0)),
        ],
        out_specs=pl.BlockSpec((BR, N), lambda i: (i, 0)),
        out_shape=jax.ShapeDtypeStruct((B, N), jnp.float32),
    )(f, x)
